# baseline (speedup 1.0000x reference)
import jax
import jax.numpy as jnp
from jax.experimental import pallas as pl
from jax.experimental.pallas import tpu as pltpu


def _router_kernel(x0_ref, gw_ref, ids_ref):
    e = gw_ref.shape[0]
    x0 = x0_ref[...]
    logits = jax.lax.dot_general(
        x0, gw_ref[...], (((1,), (1,)), ((), ())),
        preferred_element_type=jnp.float32)
    iota = jax.lax.broadcasted_iota(jnp.int32, (1, e), 1)
    m0 = jnp.max(logits, axis=1, keepdims=True)
    i0 = jnp.min(jnp.where(logits == m0, iota, e), axis=1, keepdims=True)
    masked = jnp.where(iota == i0, -jnp.inf, logits)
    m1 = jnp.max(masked, axis=1, keepdims=True)
    i1 = jnp.min(jnp.where(masked == m1, iota, e), axis=1, keepdims=True)
    ids_ref[...] = jnp.concatenate([i0, i1], axis=1)


def _moe_kernel(ids_ref, x_ref, gw_ref, w0_ref, w1_ref, b0_ref, b1_ref,
                cw_ref, o_ref):
    del ids_ref
    e = gw_ref.shape[0]
    x = x_ref[...]
    logits = jax.lax.dot_general(
        x, gw_ref[...], (((1,), (1,)), ((), ())),
        preferred_element_type=jnp.float32)
    m = jnp.max(logits, axis=1, keepdims=True)
    ex = jnp.exp(logits - m)
    probs = ex / jnp.sum(ex, axis=1, keepdims=True)
    iota = jax.lax.broadcasted_iota(jnp.int32, probs.shape, 1)
    w_top1 = jnp.max(probs, axis=1, keepdims=True)
    idx1 = jnp.min(jnp.where(probs == w_top1, iota, e), axis=1, keepdims=True)
    masked = jnp.where(iota == idx1, -1.0, probs)
    w_top2 = jnp.max(masked, axis=1, keepdims=True)

    pre0 = jax.lax.dot_general(
        x, w0_ref[0], (((1,), (1,)), ((), ())),
        preferred_element_type=jnp.float32) + b0_ref[0]
    pre1 = jax.lax.dot_general(
        x, w1_ref[0], (((1,), (1,)), ((), ())),
        preferred_element_type=jnp.float32) + b1_ref[0]
    acc = (pre0 * jax.nn.sigmoid(pre0) * w_top1
           + pre1 * jax.nn.sigmoid(pre1) * w_top2)

    o_ref[...] = jax.lax.dot_general(
        acc, cw_ref[...], (((1,), (1,)), ((), ())),
        preferred_element_type=jnp.float32)


def kernel(tokens, gate_w, expert_w, expert_b, combine_w):
    b, s, h = tokens.shape
    n = b * s
    e = gate_w.shape[0]
    x = tokens.reshape(n, h)

    ids = pl.pallas_call(
        _router_kernel,
        out_shape=jax.ShapeDtypeStruct((1, 2), jnp.int32),
    )(x[0:1], gate_w)
    ids = ids.reshape(2)

    t = 128
    while n % t:
        t //= 2

    out = pl.pallas_call(
        _moe_kernel,
        grid_spec=pltpu.PrefetchScalarGridSpec(
            num_scalar_prefetch=1,
            grid=(n // t,),
            in_specs=[
                pl.BlockSpec((t, h), lambda i, ids: (i, 0)),
                pl.BlockSpec((e, h), lambda i, ids: (0, 0)),
                pl.BlockSpec((1, h, h), lambda i, ids: (ids[0], 0, 0)),
                pl.BlockSpec((1, h, h), lambda i, ids: (ids[1], 0, 0)),
                pl.BlockSpec((1, 1, h), lambda i, ids: (ids[0], 0, 0)),
                pl.BlockSpec((1, 1, h), lambda i, ids: (ids[1], 0, 0)),
                pl.BlockSpec((h, h), lambda i, ids: (0, 0)),
            ],
            out_specs=pl.BlockSpec((t, h), lambda i, ids: (i, 0)),
        ),
        out_shape=jax.ShapeDtypeStruct((n, h), jnp.float32),
    )(ids, x, gate_w, expert_w, expert_w, expert_b.reshape(e, 1, h),
      expert_b.reshape(e, 1, h), combine_w)
    return out.reshape(b, s, h)


# speedup vs baseline: 1.6059x; 1.6059x over previous
import jax
import jax.numpy as jnp
from jax.experimental import pallas as pl
from jax.experimental.pallas import tpu as pltpu


def _router_kernel(x0_ref, gw_ref, ids_ref):
    e = gw_ref.shape[0]
    x0 = x0_ref[...]
    logits = jax.lax.dot_general(
        x0, gw_ref[...], (((1,), (1,)), ((), ())),
        preferred_element_type=jnp.float32)
    iota = jax.lax.broadcasted_iota(jnp.int32, (1, e), 1)
    m0 = jnp.max(logits, axis=1, keepdims=True)
    i0 = jnp.min(jnp.where(logits == m0, iota, e), axis=1, keepdims=True)
    masked = jnp.where(iota == i0, -jnp.inf, logits)
    m1 = jnp.max(masked, axis=1, keepdims=True)
    i1 = jnp.min(jnp.where(masked == m1, iota, e), axis=1, keepdims=True)
    ids_ref[...] = jnp.concatenate([i0, i1], axis=1)


def _cast_kernel(ids_ref, ew_ref, wb_ref):
    del ids_ref
    wb_ref[...] = ew_ref[...].astype(jnp.bfloat16)


def _moe_kernel(ids_ref, x_ref, gw_ref, w0_ref, w1_ref, b0_ref, b1_ref,
                cw_ref, o_ref):
    del ids_ref
    e = gw_ref.shape[0]
    x = x_ref[...]
    logits = jax.lax.dot_general(
        x, gw_ref[...], (((1,), (1,)), ((), ())),
        preferred_element_type=jnp.float32)
    m = jnp.max(logits, axis=1, keepdims=True)
    ex = jnp.exp(logits - m)
    probs = ex / jnp.sum(ex, axis=1, keepdims=True)
    iota = jax.lax.broadcasted_iota(jnp.int32, probs.shape, 1)
    w_top1 = jnp.max(probs, axis=1, keepdims=True)
    idx1 = jnp.min(jnp.where(probs == w_top1, iota, e), axis=1, keepdims=True)
    masked = jnp.where(iota == idx1, -1.0, probs)
    w_top2 = jnp.max(masked, axis=1, keepdims=True)

    xb = x.astype(jnp.bfloat16)
    pre0 = jax.lax.dot_general(
        xb, w0_ref[0], (((1,), (1,)), ((), ())),
        preferred_element_type=jnp.float32) + b0_ref[0]
    pre1 = jax.lax.dot_general(
        xb, w1_ref[0], (((1,), (1,)), ((), ())),
        preferred_element_type=jnp.float32) + b1_ref[0]
    acc = (pre0 * jax.nn.sigmoid(pre0) * w_top1
           + pre1 * jax.nn.sigmoid(pre1) * w_top2)

    o_ref[...] = jax.lax.dot_general(
        acc.astype(jnp.bfloat16), cw_ref[...], (((1,), (1,)), ((), ())),
        preferred_element_type=jnp.float32)


def kernel(tokens, gate_w, expert_w, expert_b, combine_w):
    b, s, h = tokens.shape
    n = b * s
    e = gate_w.shape[0]
    x = tokens.reshape(n, h)

    ids = pl.pallas_call(
        _router_kernel,
        out_shape=jax.ShapeDtypeStruct((1, 2), jnp.int32),
    )(x[0:1], gate_w)
    ids = ids.reshape(2)

    wb = pl.pallas_call(
        _cast_kernel,
        grid_spec=pltpu.PrefetchScalarGridSpec(
            num_scalar_prefetch=1,
            grid=(2,),
            in_specs=[pl.BlockSpec((1, h, h), lambda i, ids: (ids[i], 0, 0))],
            out_specs=pl.BlockSpec((1, h, h), lambda i, ids: (i, 0, 0)),
        ),
        out_shape=jax.ShapeDtypeStruct((2, h, h), jnp.bfloat16),
    )(ids, expert_w)
    cb = combine_w.astype(jnp.bfloat16)

    t = 256
    while n % t:
        t //= 2

    out = pl.pallas_call(
        _moe_kernel,
        grid_spec=pltpu.PrefetchScalarGridSpec(
            num_scalar_prefetch=1,
            grid=(n // t,),
            in_specs=[
                pl.BlockSpec((t, h), lambda i, ids: (i, 0)),
                pl.BlockSpec((e, h), lambda i, ids: (0, 0)),
                pl.BlockSpec((1, h, h), lambda i, ids: (0, 0, 0)),
                pl.BlockSpec((1, h, h), lambda i, ids: (1, 0, 0)),
                pl.BlockSpec((1, 1, h), lambda i, ids: (ids[0], 0, 0)),
                pl.BlockSpec((1, 1, h), lambda i, ids: (ids[1], 0, 0)),
                pl.BlockSpec((h, h), lambda i, ids: (0, 0)),
            ],
            out_specs=pl.BlockSpec((t, h), lambda i, ids: (i, 0)),
        ),
        out_shape=jax.ShapeDtypeStruct((n, h), jnp.float32),
    )(ids, x, gate_w, wb, wb, expert_b.reshape(e, 1, h),
      expert_b.reshape(e, 1, h), cb)
    return out.reshape(b, s, h)
